# Initial kernel scaffold; baseline (speedup 1.0000x reference)
import jax
import jax.numpy as jnp
from jax.experimental import pallas as pl


def kernel(x, pos_table):
    raise NotImplementedError("write your pallas kernel here")


# speedup vs baseline: 1.6693x; 1.6693x over previous
import jax
import jax.numpy as jnp
from jax.experimental import pallas as pl


_TS = 1024


def _add_kernel(x_ref, p_ref, o_ref):
    o_ref[...] = x_ref[...] + p_ref[...]


def kernel(x, pos_table):
    B, S, D = x.shape
    ts = _TS
    grid = (S // ts, B)
    return pl.pallas_call(
        _add_kernel,
        grid=grid,
        in_specs=[
            pl.BlockSpec((1, ts, D), lambda s, b: (b, s, 0)),
            pl.BlockSpec((ts, D), lambda s, b: (s, 0)),
        ],
        out_specs=pl.BlockSpec((1, ts, D), lambda s, b: (b, s, 0)),
        out_shape=jax.ShapeDtypeStruct((B, S, D), x.dtype),
    )(x, pos_table)


# speedup vs baseline: 1.7398x; 1.0422x over previous
import jax
import jax.numpy as jnp
from jax.experimental import pallas as pl


_TS = 2048


def _add_kernel(x_ref, p_ref, o_ref):
    o_ref[...] = x_ref[...] + p_ref[...]


def kernel(x, pos_table):
    B, S, D = x.shape
    ts = _TS
    grid = (S // ts, B)
    return pl.pallas_call(
        _add_kernel,
        grid=grid,
        in_specs=[
            pl.BlockSpec((1, ts, D), lambda s, b: (b, s, 0)),
            pl.BlockSpec((ts, D), lambda s, b: (s, 0)),
        ],
        out_specs=pl.BlockSpec((1, ts, D), lambda s, b: (b, s, 0)),
        out_shape=jax.ShapeDtypeStruct((B, S, D), x.dtype),
    )(x, pos_table)
